# baseline (speedup 1.0000x reference)
import functools
import math

import jax
import jax.numpy as jnp
from jax import lax
from jax.experimental import pallas as pl
from jax.experimental.pallas import tpu as pltpu
from jax.experimental.pallas import tpu_sc as plsc

D_MODEL = 64
VOCAB = 1000000
BATCH = 4096
HIST = 200
SCALE = math.sqrt(D_MODEL)

NC = 2
NS = 16
NW = NC * NS

B_TOT = BATCH * HIST
B_PER_W = B_TOT // NW
G = 128
N_GROUPS = B_PER_W // G


def _body(idx_hbm, lut_hbm, out_hbm, idx_v, buf, sem):
    cid = lax.axis_index("c")
    sid = lax.axis_index("s")
    wid = sid * NC + cid
    base = wid * B_PER_W

    pltpu.sync_copy(idx_hbm.at[wid], idx_v)

    eight = jnp.full((16,), SCALE, jnp.float32)

    @pl.loop(0, N_GROUPS)
    def _(g):
        pltpu.async_copy(lut_hbm.at[idx_v.at[g]], buf, sem).wait()
        @pl.loop(0, G)
        def _(i):
            for j in range(D_MODEL // 16):
                sl = buf[i, pl.ds(j * 16, 16)]
                buf[i, pl.ds(j * 16, 16)] = sl * eight
        pltpu.sync_copy(buf, out_hbm.at[pl.ds(base + g * G, G)])


@functools.partial(jax.jit, static_argnames=())
def _run(idx, lut):
    mesh = plsc.VectorSubcoreMesh(core_axis_name="c", subcore_axis_name="s")
    f = pl.kernel(
        _body,
        out_type=jax.ShapeDtypeStruct((B_TOT, D_MODEL), jnp.float32),
        mesh=mesh,
        scratch_types=[
            pltpu.VMEM((N_GROUPS, G), jnp.int32),
            pltpu.VMEM((G, D_MODEL), jnp.float32),
            pltpu.SemaphoreType.DMA,
        ],
        compiler_params=pltpu.CompilerParams(use_tc_tiling_on_sc=False),
    )
    return f(idx, lut)


def kernel(x, lut):
    idx = x.astype(jnp.int32).reshape(NW, N_GROUPS, G)
    out = _run(idx, lut)
    return out.reshape(BATCH, HIST, D_MODEL)


# speedup vs baseline: 1.2010x; 1.2010x over previous
import functools
import math

import jax
import jax.numpy as jnp
from jax import lax
from jax.experimental import pallas as pl
from jax.experimental.pallas import tpu as pltpu
from jax.experimental.pallas import tpu_sc as plsc

D_MODEL = 64
VOCAB = 1000000
BATCH = 4096
HIST = 200
SCALE = math.sqrt(D_MODEL)

NC = 2
NS = 16
NW = NC * NS

B_TOT = BATCH * HIST
B_PER_W = B_TOT // NW
G = 128
N_GROUPS = B_PER_W // G
NBUF = 4


def _body(idx_hbm, lut_hbm, out_hbm, idx_v, bufs, gsem, osem):
    cid = lax.axis_index("c")
    sid = lax.axis_index("s")
    wid = sid * NC + cid
    base = wid * B_PER_W

    pltpu.sync_copy(idx_hbm.at[wid], idx_v)

    eight = jnp.full((16,), SCALE, jnp.float32)

    @pl.loop(0, N_GROUPS, step=NBUF)
    def _(g0):
        for b in range(NBUF):
            @pl.when(g0 > 0)
            def _():
                pltpu.make_async_copy(
                    bufs.at[b], out_hbm.at[pl.ds(base, G)], osem.at[b]
                ).wait()
            pltpu.async_copy(lut_hbm.at[idx_v.at[g0 + b]], bufs.at[b], gsem.at[b])
        for b in range(NBUF):
            pltpu.make_async_copy(
                lut_hbm.at[idx_v.at[g0 + b]], bufs.at[b], gsem.at[b]
            ).wait()

            @pl.loop(0, G, unroll=8)
            def _(i):
                for j in range(D_MODEL // 16):
                    sl = bufs[b, i, pl.ds(j * 16, 16)]
                    bufs[b, i, pl.ds(j * 16, 16)] = sl * eight

            pltpu.async_copy(
                bufs.at[b], out_hbm.at[pl.ds(base + (g0 + b) * G, G)], osem.at[b]
            )

    for b in range(NBUF):
        pltpu.make_async_copy(
            bufs.at[b], out_hbm.at[pl.ds(base, G)], osem.at[b]
        ).wait()


@jax.jit
def _run(idx, lut):
    mesh = plsc.VectorSubcoreMesh(core_axis_name="c", subcore_axis_name="s")
    f = pl.kernel(
        _body,
        out_type=jax.ShapeDtypeStruct((B_TOT, D_MODEL), jnp.float32),
        mesh=mesh,
        scratch_types=[
            pltpu.VMEM((N_GROUPS, G), jnp.int32),
            pltpu.VMEM((NBUF, G, D_MODEL), jnp.float32),
            pltpu.SemaphoreType.DMA((NBUF,)),
            pltpu.SemaphoreType.DMA((NBUF,)),
        ],
        compiler_params=pltpu.CompilerParams(use_tc_tiling_on_sc=False),
    )
    return f(idx, lut)


def kernel(x, lut):
    idx = x.astype(jnp.int32).reshape(NW, N_GROUPS, G)
    out = _run(idx, lut)
    return out.reshape(BATCH, HIST, D_MODEL)
